# baseline (speedup 1.0000x reference)
import functools

import jax
import jax.numpy as jnp
from jax import lax
from jax.experimental import pallas as pl
from jax.experimental.pallas import tpu as pltpu
from jax.experimental.pallas import tpu_sc as plsc

N = 10000
D = 128
E = 320000

NC = 2
NS = 16
NW = NC * NS
NPAD = 10240
ROWS_PER_TILE = NPAD // NS
CH = 64
EPW = E // NW
NCHUNK = 160
EPW_PAD = NCHUNK * CH


NQ = 4
NSUP = NCHUNK // NQ


def _sc_aggregate_body(y_hbm, ei_hbm, out_hbm, acc_sh,
                       idx_v, r0_v, r1_v, r2_v, r3_v,
                       sem_0, sem_1, sem_2, sem_3):
    c = lax.axis_index("c")
    s = lax.axis_index("s")
    wid = s * NC + c
    row0 = s * ROWS_PER_TILE

    rows = [r0_v, r1_v, r2_v, r3_v]
    sems = [sem_0, sem_1, sem_2, sem_3]

    def _zero_row(i, carry):
        for j in range(D // 16):
            r0_v[i, pl.ds(j * 16, 16)] = jnp.zeros((16,), jnp.float32)
        return carry
    lax.fori_loop(0, CH, _zero_row, 0)
    for b in range(ROWS_PER_TILE // CH):
        pltpu.sync_copy(r0_v.at[pl.ds(0, CH)],
                        acc_sh.at[pl.ds(row0 + b * CH, CH)])
    plsc.subcore_barrier()

    def _super(k, carry):
        pltpu.sync_copy(ei_hbm.at[wid, k], idx_v)
        copies = [pltpu.async_copy(y_hbm.at[idx_v.at[q, 0]], rows[q], sems[q])
                  for q in range(NQ)]
        for q in range(NQ):
            copies[q].wait()
            pltpu.sync_copy(rows[q], acc_sh.at[idx_v.at[q, 1]], add=True)
        return carry
    lax.fori_loop(0, NSUP, _super, 0)
    plsc.subcore_barrier()

    pltpu.sync_copy(acc_sh.at[pl.ds(row0, ROWS_PER_TILE)],
                    out_hbm.at[c, pl.ds(row0, ROWS_PER_TILE)])


@jax.jit
def _sc_aggregate(y_pad, ei_pad):
    mesh = plsc.VectorSubcoreMesh(core_axis_name="c", subcore_axis_name="s")
    return pl.kernel(
        _sc_aggregate_body,
        out_type=jax.ShapeDtypeStruct((NC, NPAD, D), jnp.float32),
        mesh=mesh,
        scratch_types=[
            pltpu.VMEM_SHARED((NPAD, D), jnp.float32),
            pltpu.VMEM((NQ, 2, CH), jnp.int32),
            pltpu.VMEM((CH, D), jnp.float32),
            pltpu.VMEM((CH, D), jnp.float32),
            pltpu.VMEM((CH, D), jnp.float32),
            pltpu.VMEM((CH, D), jnp.float32),
            pltpu.SemaphoreType.DMA,
            pltpu.SemaphoreType.DMA,
            pltpu.SemaphoreType.DMA,
            pltpu.SemaphoreType.DMA,
        ],
    )(y_pad, ei_pad)


def _tc_pre_body(h_ref, w_ref, y_ref):
    y_ref[...] = jnp.dot(h_ref[...], w_ref[...],
                         preferred_element_type=jnp.float32)


@jax.jit
def _tc_pre(h, w):
    return pl.pallas_call(
        _tc_pre_body,
        out_shape=jax.ShapeDtypeStruct((NPAD, D), jnp.float32),
    )(h, w)


def _tc_post_body(p_ref, h_ref, wroot_ref, b_ref, wnext_ref, h_out, y_out):
    agg = p_ref[0] + p_ref[1]
    pre = agg + jnp.dot(h_ref[...], wroot_ref[...],
                        preferred_element_type=jnp.float32) + b_ref[...]
    h = jax.nn.sigmoid(pre)
    h_out[...] = h
    y_out[...] = jnp.dot(h, wnext_ref[...], preferred_element_type=jnp.float32)


@jax.jit
def _tc_post(p, h_prev, w_root, b, w_next):
    return pl.pallas_call(
        _tc_post_body,
        out_shape=(jax.ShapeDtypeStruct((NPAD, D), jnp.float32),
                   jax.ShapeDtypeStruct((NPAD, D), jnp.float32)),
    )(p, h_prev, w_root, b.reshape(1, D), w_next)


def _tc_final_body(p_ref, h_ref, wroot_ref, b_ref, out_ref):
    agg = p_ref[0] + p_ref[1]
    out_ref[...] = agg + jnp.dot(h_ref[...], wroot_ref[...],
                                 preferred_element_type=jnp.float32) + b_ref[...]


@jax.jit
def _tc_final(p, h_prev, w_root, b):
    return pl.pallas_call(
        _tc_final_body,
        out_shape=jax.ShapeDtypeStruct((NPAD, D), jnp.float32),
    )(p, h_prev, w_root, b.reshape(1, D))


def kernel(x, edge_index, W_in_rel, b_in_rel, W_in_root,
           W_med_rel, b_med_rel, W_med_root,
           W_out_rel, b_out_rel, W_out_root):
    src = edge_index[0].astype(jnp.int32).reshape(NW, EPW)
    dst = edge_index[1].astype(jnp.int32).reshape(NW, EPW)
    src_pad = jnp.pad(src, ((0, 0), (0, EPW_PAD - EPW))).reshape(NW, NSUP, NQ, CH)
    dst_pad = jnp.pad(dst, ((0, 0), (0, EPW_PAD - EPW)),
                      constant_values=NPAD - 1).reshape(NW, NSUP, NQ, CH)
    ei_pad = jnp.stack([src_pad, dst_pad], axis=3)
    x_pad = jnp.pad(x, ((0, NPAD - N), (0, 0)))

    y1 = _tc_pre(x_pad, W_in_rel)
    p1 = _sc_aggregate(y1, ei_pad)
    h1, y2 = _tc_post(p1, x_pad, W_in_root, b_in_rel, W_med_rel)
    p2 = _sc_aggregate(y2, ei_pad)
    h2, y3 = _tc_post(p2, h1, W_med_root, b_med_rel, W_med_rel)
    p3 = _sc_aggregate(y3, ei_pad)
    h3, y4 = _tc_post(p3, h2, W_med_root, b_med_rel, W_out_rel)
    p4 = _sc_aggregate(y4, ei_pad)
    out_pad = _tc_final(p4, h3, W_out_root, b_out_rel)
    return out_pad[:N]


# speedup vs baseline: 1.2452x; 1.2452x over previous
import functools

import jax
import jax.numpy as jnp
from jax import lax
from jax.experimental import pallas as pl
from jax.experimental.pallas import tpu as pltpu
from jax.experimental.pallas import tpu_sc as plsc

N = 10000
D = 128
E = 320000

NC = 2
NS = 16
NW = NC * NS
NPAD = 10240
ROWS_PER_TILE = NPAD // NS
CH = 128
EPW = E // NW
NCHUNK = 79
EPW_PAD = NCHUNK * CH


def _sc_aggregate_body(y_hbm, src_hbm, dst_hbm, out_hbm,
                       acc_sh, src_v, dst_v, rows_v, zrows_v, sem):
    c = lax.axis_index("c")
    s = lax.axis_index("s")
    wid = s * NC + c

    def _zero_row(i, carry):
        for j in range(D // 16):
            zrows_v[i, pl.ds(j * 16, 16)] = jnp.zeros((16,), jnp.float32)
        return carry
    lax.fori_loop(0, CH, _zero_row, 0)
    for b in range(ROWS_PER_TILE // CH):
        pltpu.sync_copy(zrows_v, acc_sh.at[pl.ds(s * ROWS_PER_TILE + b * CH, CH)])
    plsc.subcore_barrier()

    base = wid * EPW_PAD

    def _chunk(i, carry):
        off = pl.multiple_of(base + i * CH, CH)
        pltpu.sync_copy(src_hbm.at[pl.ds(off, CH)], src_v)
        pltpu.sync_copy(dst_hbm.at[pl.ds(off, CH)], dst_v)
        pltpu.async_copy(y_hbm.at[src_v], rows_v, sem).wait()
        pltpu.sync_copy(rows_v, acc_sh.at[dst_v], add=True)
        return carry
    lax.fori_loop(0, NCHUNK, _chunk, 0)
    plsc.subcore_barrier()

    row0 = s * ROWS_PER_TILE
    pltpu.sync_copy(acc_sh.at[pl.ds(row0, ROWS_PER_TILE)],
                    out_hbm.at[c, pl.ds(row0, ROWS_PER_TILE)])


@jax.jit
def _sc_aggregate(y_pad, src_pad, dst_pad):
    mesh = plsc.VectorSubcoreMesh(core_axis_name="c", subcore_axis_name="s")
    return pl.kernel(
        _sc_aggregate_body,
        out_type=jax.ShapeDtypeStruct((NC, NPAD, D), jnp.float32),
        mesh=mesh,
        scratch_types=[
            pltpu.VMEM_SHARED((NPAD, D), jnp.float32),
            pltpu.VMEM((CH,), jnp.int32),
            pltpu.VMEM((CH,), jnp.int32),
            pltpu.VMEM((CH, D), jnp.float32),
            pltpu.VMEM((CH, D), jnp.float32),
            pltpu.SemaphoreType.DMA,
        ],
    )(y_pad, src_pad, dst_pad)


def _tc_pre_body(h_ref, w_ref, y_ref):
    y_ref[...] = jnp.dot(h_ref[...], w_ref[...],
                         preferred_element_type=jnp.float32)


@jax.jit
def _tc_pre(h, w):
    return pl.pallas_call(
        _tc_pre_body,
        out_shape=jax.ShapeDtypeStruct((NPAD, D), jnp.float32),
    )(h, w)


def _tc_post_body(p_ref, h_ref, wroot_ref, b_ref, wnext_ref, h_out, y_out):
    agg = p_ref[0] + p_ref[1]
    pre = agg + jnp.dot(h_ref[...], wroot_ref[...],
                        preferred_element_type=jnp.float32) + b_ref[...]
    h = jax.nn.sigmoid(pre)
    h_out[...] = h
    y_out[...] = jnp.dot(h, wnext_ref[...], preferred_element_type=jnp.float32)


@jax.jit
def _tc_post(p, h_prev, w_root, b, w_next):
    return pl.pallas_call(
        _tc_post_body,
        out_shape=(jax.ShapeDtypeStruct((NPAD, D), jnp.float32),
                   jax.ShapeDtypeStruct((NPAD, D), jnp.float32)),
    )(p, h_prev, w_root, b.reshape(1, D), w_next)


def _tc_final_body(p_ref, h_ref, wroot_ref, b_ref, out_ref):
    agg = p_ref[0] + p_ref[1]
    out_ref[...] = agg + jnp.dot(h_ref[...], wroot_ref[...],
                                 preferred_element_type=jnp.float32) + b_ref[...]


@jax.jit
def _tc_final(p, h_prev, w_root, b):
    return pl.pallas_call(
        _tc_final_body,
        out_shape=jax.ShapeDtypeStruct((NPAD, D), jnp.float32),
    )(p, h_prev, w_root, b.reshape(1, D))


def kernel(x, edge_index, W_in_rel, b_in_rel, W_in_root,
           W_med_rel, b_med_rel, W_med_root,
           W_out_rel, b_out_rel, W_out_root):
    src = edge_index[0].astype(jnp.int32).reshape(NW, EPW)
    dst = edge_index[1].astype(jnp.int32).reshape(NW, EPW)
    src_pad = jnp.pad(src, ((0, 0), (0, EPW_PAD - EPW))).reshape(-1)
    dst_pad = jnp.pad(dst, ((0, 0), (0, EPW_PAD - EPW)),
                      constant_values=NPAD - 1).reshape(-1)
    x_pad = jnp.pad(x, ((0, NPAD - N), (0, 0)))

    y1 = _tc_pre(x_pad, W_in_rel)
    p1 = _sc_aggregate(y1, src_pad, dst_pad)
    h1, y2 = _tc_post(p1, x_pad, W_in_root, b_in_rel, W_med_rel)
    p2 = _sc_aggregate(y2, src_pad, dst_pad)
    h2, y3 = _tc_post(p2, h1, W_med_root, b_med_rel, W_med_rel)
    p3 = _sc_aggregate(y3, src_pad, dst_pad)
    h3, y4 = _tc_post(p3, h2, W_med_root, b_med_rel, W_out_rel)
    p4 = _sc_aggregate(y4, src_pad, dst_pad)
    out_pad = _tc_final(p4, h3, W_out_root, b_out_rel)
    return out_pad[:N]
